# baseline (speedup 1.0000x reference)
import functools

import jax
import jax.numpy as jnp
from jax import lax
from jax.experimental import pallas as pl
from jax.experimental.pallas import tpu as pltpu
from jax.experimental.pallas import tpu_sc as plsc

N_NODES = 10000
N_EDGES = 320000
D_FEAT = 128

NC = 2
NS = 16
NW = NC * NS

CHUNK = 128
NMAIN = N_EDGES // (NW * CHUNK)
REM = (N_EDGES - NW * NMAIN * CHUNK) // NW
NBUF = 3

ZN = 5
OROWS = 624
OTAIL = N_NODES - OROWS * NS


def _sc_scatter_gather(x, edge_index):
  mesh = plsc.VectorSubcoreMesh(core_axis_name="c", subcore_axis_name="s")

  @functools.partial(
      pl.kernel,
      out_type=jax.ShapeDtypeStruct((NC, N_NODES, D_FEAT), jnp.float32),
      mesh=mesh,
      scratch_types=[
          [pltpu.VMEM((2, CHUNK), jnp.int32)] * NBUF,
          [pltpu.VMEM((CHUNK, D_FEAT), jnp.float32)] * NBUF,
          pltpu.VMEM((2, NW * REM), jnp.int32),
          pltpu.VMEM((REM,), jnp.int32),
          pltpu.VMEM_SHARED((N_NODES, D_FEAT), jnp.float32),
          [pltpu.SemaphoreType.DMA] * NBUF,
          [pltpu.SemaphoreType.DMA] * NBUF,
          [pltpu.SemaphoreType.DMA] * NBUF,
          [pltpu.SemaphoreType.DMA] * NBUF,
          pltpu.SemaphoreType.DMA,
      ],
  )
  def k(x_hbm, ei_hbm, out_hbm, idxv, rows, exv, dex, acc,
        isem, gsem, hsem, ssem, xsem):
    cid = lax.axis_index("c")
    sid = lax.axis_index("s")
    tile = cid * NS + sid
    ebase = NMAIN * tile * CHUNK

    def load_idx(c, b):
      pltpu.async_copy(ei_hbm.at[:, pl.ds(ebase + c * CHUNK, CHUNK)],
                       idxv[b], isem[b])

    H = CHUNK // 2

    def gather(c, b):
      pltpu.make_async_copy(ei_hbm.at[:, pl.ds(0, CHUNK)], idxv[b],
                            isem[b]).wait()
      pltpu.async_copy(x_hbm.at[idxv[b].at[0, pl.ds(0, H)]],
                       rows[b].at[pl.ds(0, H)], gsem[b])
      pltpu.async_copy(x_hbm.at[idxv[b].at[0, pl.ds(H, H)]],
                       rows[b].at[pl.ds(H, H)], hsem[b])

    for c in range(NBUF):
      load_idx(c, c)
    for c in range(NBUF - 1):
      gather(c, c)
    pltpu.async_copy(ei_hbm.at[:, pl.ds(NW * NMAIN * CHUNK, NW * REM)],
                     exv, xsem)

    zb = NBUF - 1

    @pl.loop(0, CHUNK)
    def _zrow(i):
      for j in range(D_FEAT // 16):
        rows[zb][i, pl.ds(j * 16, 16)] = jnp.zeros((16,), jnp.float32)

    @pl.loop(0, ZN)
    def _zacc(z):
      pltpu.sync_copy(rows[zb], acc.at[pl.ds(sid * OROWS + z * CHUNK, CHUNK)])

    plsc.subcore_barrier()

    def _do_chunk(ci, b, bo, prefetch=True):
      pltpu.make_async_copy(x_hbm.at[idxv[b].at[0, pl.ds(0, H)]],
                            rows[b].at[pl.ds(0, H)], gsem[b]).wait()
      pltpu.make_async_copy(x_hbm.at[idxv[b].at[0, pl.ds(H, H)]],
                            rows[b].at[pl.ds(H, H)], hsem[b]).wait()
      pltpu.async_copy(rows[b], acc.at[idxv[b].at[1]], ssem[b], add=True)

      @pl.when(ci > 0)
      def _drain_prev():
        pltpu.make_async_copy(rows[bo], acc.at[idxv[bo].at[1]],
                              ssem[bo]).wait()

      if prefetch:
        @pl.when(ci + NBUF < NMAIN)
        def _prefetch_idx():
          load_idx(ci + NBUF, b)

        @pl.when(ci + NBUF - 1 < NMAIN)
        def _prefetch_gather():
          gather(ci + NBUF - 1, bo)

    @pl.loop(0, NMAIN, step=NBUF)
    def _group(ci0):
      for b in range(NBUF):
        _do_chunk(ci0 + b, b, (b + NBUF - 1) % NBUF)

    bl = (NMAIN - 1) % NBUF
    pltpu.make_async_copy(rows[bl], acc.at[idxv[bl].at[1]], ssem[bl]).wait()

    pltpu.make_async_copy(ei_hbm.at[:, pl.ds(0, NW * REM)], exv, xsem).wait()
    dex[...] = exv[1, pl.ds(tile * REM, REM)]
    pltpu.async_copy(x_hbm.at[exv.at[0, pl.ds(tile * REM, REM)]],
                     rows[0].at[pl.ds(0, REM)], gsem[0]).wait()
    pltpu.sync_copy(rows[0].at[pl.ds(0, REM)], acc.at[dex], add=True)

    plsc.subcore_barrier()
    pltpu.sync_copy(acc.at[pl.ds(sid * OROWS, OROWS)],
                    out_hbm.at[cid, pl.ds(sid * OROWS, OROWS)])

    @pl.when(sid == NS - 1)
    def _tail():
      pltpu.sync_copy(acc.at[pl.ds(OROWS * NS, OTAIL)],
                      out_hbm.at[cid, pl.ds(OROWS * NS, OTAIL)])

  return k(x, edge_index)


def _tc_add(partial):
  def body(a_ref, b_ref, o_ref):
    o_ref[...] = a_ref[0] + b_ref[0]

  blk = 5000
  return pl.pallas_call(
      body,
      out_shape=jax.ShapeDtypeStruct((N_NODES, D_FEAT), jnp.float32),
      grid=(N_NODES // blk,),
      in_specs=[
          pl.BlockSpec((1, blk, D_FEAT), lambda i: (0, i, 0)),
          pl.BlockSpec((1, blk, D_FEAT), lambda i: (1, i, 0)),
      ],
      out_specs=pl.BlockSpec((blk, D_FEAT), lambda i: (i, 0)),
  )(partial, partial)


@jax.jit
def kernel(x, edge_index):
  partial = _sc_scatter_gather(x, edge_index)
  return _tc_add(partial)


# speedup vs baseline: 1.0057x; 1.0057x over previous
import functools

import jax
import jax.numpy as jnp
from jax import lax
from jax.experimental import pallas as pl
from jax.experimental.pallas import tpu as pltpu
from jax.experimental.pallas import tpu_sc as plsc

N_NODES = 10000
N_EDGES = 320000
D_FEAT = 128

NC = 2
NS = 16
NW = NC * NS

CHUNK = 128
NMAIN = N_EDGES // (NW * CHUNK)
REM = (N_EDGES - NW * NMAIN * CHUNK) // NW
NBUF = 3

ZN = 5
OROWS = 624
OTAIL = N_NODES - OROWS * NS


def _sc_scatter_gather(x, edge_index):
  mesh = plsc.VectorSubcoreMesh(core_axis_name="c", subcore_axis_name="s")

  @functools.partial(
      pl.kernel,
      out_type=jax.ShapeDtypeStruct((NC, N_NODES, D_FEAT), jnp.float32),
      mesh=mesh,
      scratch_types=[
          [pltpu.VMEM((2, CHUNK), jnp.int32)] * NBUF,
          [pltpu.VMEM((CHUNK, D_FEAT), jnp.float32)] * NBUF,
          pltpu.VMEM((2, NW * REM), jnp.int32),
          pltpu.VMEM((REM,), jnp.int32),
          pltpu.VMEM_SHARED((N_NODES, D_FEAT), jnp.float32),
          [pltpu.SemaphoreType.DMA] * NBUF,
          [pltpu.SemaphoreType.DMA] * NBUF,
          [pltpu.SemaphoreType.DMA] * NBUF,
          [pltpu.SemaphoreType.DMA] * NBUF,
          pltpu.SemaphoreType.DMA,
      ],
  )
  def k(x_hbm, ei_hbm, out_hbm, idxv, rows, exv, dex, acc,
        isem, gsem, hsem, ssem, xsem):
    cid = lax.axis_index("c")
    sid = lax.axis_index("s")
    tile = cid * NS + sid
    ebase = NMAIN * tile * CHUNK

    def load_idx(c, b):
      pltpu.async_copy(ei_hbm.at[:, pl.ds(ebase + c * CHUNK, CHUNK)],
                       idxv[b], isem[b])

    H = CHUNK // 2

    def gather(c, b):
      pltpu.make_async_copy(ei_hbm.at[:, pl.ds(0, CHUNK)], idxv[b],
                            isem[b]).wait()
      pltpu.async_copy(x_hbm.at[idxv[b].at[0, pl.ds(0, H)]],
                       rows[b].at[pl.ds(0, H)], gsem[b])
      pltpu.async_copy(x_hbm.at[idxv[b].at[0, pl.ds(H, H)]],
                       rows[b].at[pl.ds(H, H)], hsem[b])

    for c in range(NBUF):
      load_idx(c, c)
    for c in range(NBUF - 1):
      gather(c, c)
    pltpu.async_copy(ei_hbm.at[:, pl.ds(NW * NMAIN * CHUNK, NW * REM)],
                     exv, xsem)

    zb = NBUF - 1

    @pl.loop(0, CHUNK)
    def _zrow(i):
      for j in range(D_FEAT // 16):
        rows[zb][i, pl.ds(j * 16, 16)] = jnp.zeros((16,), jnp.float32)

    @pl.loop(0, ZN)
    def _zacc(z):
      pltpu.sync_copy(rows[zb], acc.at[pl.ds(sid * OROWS + z * CHUNK, CHUNK)])

    plsc.subcore_barrier()

    def _do_chunk(ci, b, bo, prefetch=True):
      pltpu.make_async_copy(x_hbm.at[idxv[b].at[0, pl.ds(0, H)]],
                            rows[b].at[pl.ds(0, H)], gsem[b]).wait()
      pltpu.make_async_copy(x_hbm.at[idxv[b].at[0, pl.ds(H, H)]],
                            rows[b].at[pl.ds(H, H)], hsem[b]).wait()
      pltpu.async_copy(rows[b], acc.at[idxv[b].at[1]], ssem[b], add=True)

      @pl.when(ci > 0)
      def _drain_prev():
        pltpu.make_async_copy(rows[bo], acc.at[idxv[bo].at[1]],
                              ssem[bo]).wait()

      if prefetch:
        @pl.when(ci + NBUF < NMAIN)
        def _prefetch_idx():
          load_idx(ci + NBUF, b)

        @pl.when(ci + NBUF - 1 < NMAIN)
        def _prefetch_gather():
          gather(ci + NBUF - 1, bo)

    @pl.loop(0, NMAIN, step=NBUF)
    def _group(ci0):
      for b in range(NBUF):
        _do_chunk(ci0 + b, b, (b + NBUF - 1) % NBUF)

    br, bl = NMAIN % NBUF, (NMAIN - 1) % NBUF
    pltpu.make_async_copy(ei_hbm.at[:, pl.ds(0, NW * REM)], exv, xsem).wait()
    dex[...] = exv[1, pl.ds(tile * REM, REM)]
    pltpu.async_copy(x_hbm.at[exv.at[0, pl.ds(tile * REM, REM)]],
                     rows[br].at[pl.ds(0, REM)], gsem[br])
    pltpu.make_async_copy(rows[bl], acc.at[idxv[bl].at[1]], ssem[bl]).wait()
    pltpu.make_async_copy(x_hbm.at[exv.at[0, pl.ds(tile * REM, REM)]],
                          rows[br].at[pl.ds(0, REM)], gsem[br]).wait()
    pltpu.sync_copy(rows[br].at[pl.ds(0, REM)], acc.at[dex], add=True)

    plsc.subcore_barrier()
    pltpu.sync_copy(acc.at[pl.ds(sid * OROWS, OROWS)],
                    out_hbm.at[cid, pl.ds(sid * OROWS, OROWS)])

    @pl.when(sid == NS - 1)
    def _tail():
      pltpu.sync_copy(acc.at[pl.ds(OROWS * NS, OTAIL)],
                      out_hbm.at[cid, pl.ds(OROWS * NS, OTAIL)])

  return k(x, edge_index)


def _tc_add(partial):
  def body(a_ref, b_ref, o_ref):
    o_ref[...] = a_ref[0] + b_ref[0]

  blk = 5000
  return pl.pallas_call(
      body,
      out_shape=jax.ShapeDtypeStruct((N_NODES, D_FEAT), jnp.float32),
      grid=(N_NODES // blk,),
      in_specs=[
          pl.BlockSpec((1, blk, D_FEAT), lambda i: (0, i, 0)),
          pl.BlockSpec((1, blk, D_FEAT), lambda i: (1, i, 0)),
      ],
      out_specs=pl.BlockSpec((blk, D_FEAT), lambda i: (i, 0)),
  )(partial, partial)


@jax.jit
def kernel(x, edge_index):
  partial = _sc_scatter_gather(x, edge_index)
  return _tc_add(partial)
